# baseline (speedup 1.0000x reference)
import math

import jax
import jax.numpy as jnp
from jax.experimental import pallas as pl
from jax.experimental.pallas import tpu as pltpu

G = 512
A = 32
N = G * A
H = 128
F = 64
R = 12
CUT = 5.0
NAF = 13
NMF = 8
OUT = 256
NTYPES = 101
EMB = 5

GB = 32
M = GB * A
P8 = 8
AQ = A // P8
PQ = M * AQ
WL = P8 * F

_INTERPRET = False

_C0 = math.sqrt(2.0 / CUT)


def _block_kernel(x_ref, posr_ref, posc_ref, T_ref, Wn_ref, bn_ref,
                  Wh0_ref, Wr0_ref, Wuh0_ref, Wua0_ref, bu0_ref,
                  Wh1_ref, Wr1_ref, Wuh1_ref, Wua1_ref, bu1_ref,
                  Wh2_ref, Wr2_ref, Wuh2_ref, Wua2_ref, bu2_ref,
                  Wmol_ref, bmol_ref, W1g_ref, W1m_ref, bf1_ref,
                  W2_ref, bf2_ref, Wo_ref, out_ref):
    gelu = jax.nn.gelu
    f32 = jnp.float32
    i32 = jnp.int32

    xb = x_ref[...]
    poscb = posc_ref[...]

    row = jax.lax.broadcasted_iota(i32, (M, 1), 0)
    first = (row % A == 0).astype(f32)
    molx = jnp.sum((xb * first).reshape(GB, A, NAF), axis=1)
    mol = jnp.dot(molx[:, NAF - NMF:], Wmol_ref[...],
                  preferred_element_type=f32) + bmol_ref[...]

    atype = jnp.clip((xb[:, 0:1] * NTYPES).astype(i32), 0, NTYPES - 1)
    lanes = jax.lax.broadcasted_iota(i32, (M, 128), 1)
    onehot = (lanes == atype).astype(f32)
    h = gelu(jnp.dot(onehot, T_ref[...], preferred_element_type=f32)
             + jnp.dot(xb[:, 1:], Wn_ref[...], preferred_element_type=f32)
             + bn_ref[...])

    prow = jnp.broadcast_to(posr_ref[...].reshape(GB, 1, A, 3 * P8),
                            (GB, AQ, A, 3 * P8)).reshape(PQ, 3 * P8)
    pcol = jnp.broadcast_to(poscb.reshape(GB, AQ, 1, 3 * P8),
                            (GB, AQ, A, 3 * P8)).reshape(PQ, 3 * P8)
    df = prow - pcol
    sq = df * df

    ridx = jax.lax.broadcasted_iota(i32, (PQ, 1), 0)
    jo = (ridx // A) % AQ
    ii = ridx % A
    jj = P8 * jo + jax.lax.broadcasted_iota(i32, (PQ, P8), 1)
    dcol = jnp.sqrt(sq[:, 0:P8] + sq[:, P8:2 * P8] + sq[:, 2 * P8:] + 1e-12)
    adj = (dcol < CUT) & (ii != jj)
    acol = jnp.where(adj, _C0 / dcol, 0.0)
    pencol = jnp.where(adj, 0.0, -200.0)

    ra = jax.lax.broadcasted_iota(i32, (P8, P8 * R), 0)
    rb = jax.lax.broadcasted_iota(i32, (P8, P8 * R), 1)
    REPR = (ra == rb // R).astype(f32)
    drep = jnp.dot(dcol, REPR, preferred_element_type=f32)

    se0 = jax.lax.broadcasted_iota(i32, (M // P8, M), 0)
    se1 = jax.lax.broadcasted_iota(i32, (M // P8, M), 1)
    SELS = [(P8 * se0 + p == se1).astype(f32) for p in range(P8)]

    kf = ((jax.lax.broadcasted_iota(i32, (1, P8 * R), 1) % R + 1)
          .astype(f32) * (math.pi / CUT))
    theta = drep * kf
    n = jnp.round(theta * (0.5 / math.pi))
    v = theta - n * (2.0 * math.pi)
    v2 = v * v
    s = v * (0.9999994441442891 + v2 * (-0.1666651950620369 + v2 * (
        0.00833220729172304 + v2 * (-0.00019803942981621122 + v2 * (
            2.694818791282763e-06 + v2 * -2.0177080094133367e-08)))))
    samp = s * jnp.dot(acol, REPR, preferred_element_type=f32)
    saug = jnp.concatenate([samp, pencol], axis=1)


    for (Wh_ref, Wr_ref, Wuh_ref, Wua_ref, bu_ref) in (
            (Wh0_ref, Wr0_ref, Wuh0_ref, Wua0_ref, bu0_ref),
            (Wh1_ref, Wr1_ref, Wuh1_ref, Wua1_ref, bu1_ref),
            (Wh2_ref, Wr2_ref, Wuh2_ref, Wua2_ref, bu2_ref)):
        q = jnp.dot(h, Wh_ref[...], preferred_element_type=f32)
        q8 = jnp.concatenate(
            [jnp.dot(S, q, preferred_element_type=f32) for S in SELS],
            axis=1)
        qt = jnp.broadcast_to(q8.reshape(GB, AQ, 1, WL),
                              (GB, AQ, A, WL)).reshape(PQ, WL)
        z2 = jnp.dot(saug, Wr_ref[...], preferred_element_type=f32)
        xm = qt + z2
        wm = xm * (0.7978845608028654 + 0.035677408136300125 * (xm * xm))
        m = xm + xm * jnp.tanh(wm)
        mw = jnp.dot(m, Wua_ref[...], preferred_element_type=f32)
        aggw = jnp.sum(mw.reshape(GB, AQ, A, H), axis=1).reshape(M, H)
        upd = gelu(jnp.dot(h, Wuh_ref[...], preferred_element_type=f32)
                   + aggw + bu_ref[...])
        h = h + upd

    xg = jnp.sum(h.reshape(GB, A, H), axis=1) * (1.0 / A)
    z = gelu(jnp.dot(xg, W1g_ref[...], preferred_element_type=f32)
             + jnp.dot(mol, W1m_ref[...], preferred_element_type=f32)
             + bf1_ref[...])
    z = gelu(jnp.dot(z, W2_ref[...], preferred_element_type=f32) + bf2_ref[...])
    out_ref[...] = jnp.dot(z, Wo_ref[...], preferred_element_type=f32)


def kernel(x, pos, batch, ptr, aux_ind, num_graphs, atom_emb, W_node, b_node,
           Wh0, Wr0, Wu0, bu0, Wh1, Wr1, Wu1, bu1, Wh2, Wr2, Wu2, bu2,
           W_mol, b_mol, W_fc1, b_fc1, W_fc2, b_fc2, W_out):
    f32 = jnp.float32
    T = jnp.zeros((128, H), f32).at[:NTYPES].set(
        atom_emb @ W_node[:EMB])
    Wn = W_node[EMB:]
    posr = jnp.repeat(pos, P8, axis=1)
    posc = pos.reshape(G, AQ, P8, 3).transpose(0, 1, 3, 2).reshape(
        G, AQ, 3 * P8)

    def blockdiag8(W):
        Z = jnp.zeros((P8 * R + P8, WL), f32)
        for p in range(P8):
            Z = Z.at[p * R:(p + 1) * R, p * F:(p + 1) * F].set(W)
            Z = Z.at[P8 * R + p, p * F:(p + 1) * F].set(1.0)
        return Z

    row_specs = [
        pl.BlockSpec((M, NAF), lambda g: (g, 0)),
        pl.BlockSpec((M, 3 * P8), lambda g: (g, 0)),
        pl.BlockSpec((GB, AQ, 3 * P8), lambda g: (g, 0, 0)),
    ]

    full = lambda a: pl.BlockSpec(a.shape, lambda g: tuple(0 for _ in a.shape))
    tile8 = lambda Wua: jnp.tile(0.5 * Wua, (P8, 1))
    weights = [T, Wn, b_node.reshape(1, H),
               Wh0, blockdiag8(Wr0), Wu0[:H], tile8(Wu0[H:]), bu0.reshape(1, H),
               Wh1, blockdiag8(Wr1), Wu1[:H], tile8(Wu1[H:]), bu1.reshape(1, H),
               Wh2, blockdiag8(Wr2), Wu2[:H], tile8(Wu2[H:]), bu2.reshape(1, H),
               W_mol, b_mol.reshape(1, NMF),
               W_fc1[:H], W_fc1[H:], b_fc1.reshape(1, H),
               W_fc2, b_fc2.reshape(1, H), W_out]

    out = pl.pallas_call(
        _block_kernel,
        grid=(G // GB,),
        in_specs=row_specs + [full(w) for w in weights],
        out_specs=pl.BlockSpec((GB, OUT), lambda g: (g, 0)),
        out_shape=jax.ShapeDtypeStruct((G, OUT), f32),
        compiler_params=pltpu.CompilerParams(
            dimension_semantics=("parallel",)),
        interpret=_INTERPRET,
    )(x, posr, posc, *weights)
    return out


# speedup vs baseline: 1.0458x; 1.0458x over previous
import math

import jax
import jax.numpy as jnp
from jax.experimental import pallas as pl
from jax.experimental.pallas import tpu as pltpu

G = 512
A = 32
N = G * A
H = 128
F = 64
R = 12
CUT = 5.0
NAF = 13
NMF = 8
OUT = 256
NTYPES = 101
EMB = 5

GB = 32
M = GB * A
P8 = 8
AQ = A // P8
PQ = M * AQ
WL = P8 * F

_INTERPRET = False

_C0 = math.sqrt(2.0 / CUT)


def _block_kernel(x_ref, posr_ref, posc_ref, T_ref, Wn_ref, bn_ref,
                  Wh0_ref, Wr0_ref, Wuh0_ref, Wua0_ref, bu0_ref,
                  Wh1_ref, Wr1_ref, Wuh1_ref, Wua1_ref, bu1_ref,
                  Wh2_ref, Wr2_ref, Wuh2_ref, Wua2_ref, bu2_ref,
                  Wmol_ref, bmol_ref, W1g_ref, W1m_ref, bf1_ref,
                  W2_ref, bf2_ref, Wo_ref, out_ref):
    gelu = jax.nn.gelu
    f32 = jnp.float32
    i32 = jnp.int32

    xb = x_ref[...]
    poscb = posc_ref[...]

    row = jax.lax.broadcasted_iota(i32, (M, 1), 0)
    first = (row % A == 0).astype(f32)
    molx = jnp.sum((xb * first).reshape(GB, A, NAF), axis=1)
    mol = jnp.dot(molx[:, NAF - NMF:], Wmol_ref[...],
                  preferred_element_type=f32) + bmol_ref[...]

    atype = jnp.clip((xb[:, 0:1] * NTYPES).astype(i32), 0, NTYPES - 1)
    lanes = jax.lax.broadcasted_iota(i32, (M, 128), 1)
    onehot = (lanes == atype).astype(f32)
    h = gelu(jnp.dot(onehot, T_ref[...], preferred_element_type=f32)
             + jnp.dot(xb[:, 1:], Wn_ref[...], preferred_element_type=f32)
             + bn_ref[...])

    prow = jnp.broadcast_to(posr_ref[...].reshape(1, M, 3 * P8),
                            (AQ, M, 3 * P8)).reshape(PQ, 3 * P8)
    pcol = jnp.broadcast_to(poscb.reshape(AQ, GB, 1, 3 * P8),
                            (AQ, GB, A, 3 * P8)).reshape(PQ, 3 * P8)
    df = prow - pcol
    sq = df * df

    ridx = jax.lax.broadcasted_iota(i32, (PQ, 1), 0)
    jo = ridx // M
    ii = ridx % A
    jj = P8 * jo + jax.lax.broadcasted_iota(i32, (PQ, P8), 1)
    dcol = jnp.sqrt(sq[:, 0:P8] + sq[:, P8:2 * P8] + sq[:, 2 * P8:] + 1e-12)
    adj = (dcol < CUT) & (ii != jj)
    acol = jnp.where(adj, _C0 / dcol, 0.0)
    pencol = jnp.where(adj, 0.0, -200.0)

    ra = jax.lax.broadcasted_iota(i32, (P8, P8 * R), 0)
    rb = jax.lax.broadcasted_iota(i32, (P8, P8 * R), 1)
    REPR = (ra == rb // R).astype(f32)
    drep = jnp.dot(dcol, REPR, preferred_element_type=f32)

    se0 = jax.lax.broadcasted_iota(i32, (M // P8, M), 0)
    se1 = jax.lax.broadcasted_iota(i32, (M // P8, M), 1)
    SELS = [(A * (se0 % GB) + P8 * (se0 // GB) + p == se1).astype(f32)
            for p in range(P8)]

    kf = ((jax.lax.broadcasted_iota(i32, (1, P8 * R), 1) % R + 1)
          .astype(f32) * (math.pi / CUT))
    theta = drep * kf
    n = jnp.round(theta * (0.5 / math.pi))
    v = theta - n * (2.0 * math.pi)
    v2 = v * v
    s = v * (0.9999994441442891 + v2 * (-0.1666651950620369 + v2 * (
        0.00833220729172304 + v2 * (-0.00019803942981621122 + v2 * (
            2.694818791282763e-06 + v2 * -2.0177080094133367e-08)))))
    samp = s * jnp.dot(acol, REPR, preferred_element_type=f32)
    saug = jnp.concatenate([samp, pencol], axis=1)


    for (Wh_ref, Wr_ref, Wuh_ref, Wua_ref, bu_ref) in (
            (Wh0_ref, Wr0_ref, Wuh0_ref, Wua0_ref, bu0_ref),
            (Wh1_ref, Wr1_ref, Wuh1_ref, Wua1_ref, bu1_ref),
            (Wh2_ref, Wr2_ref, Wuh2_ref, Wua2_ref, bu2_ref)):
        q = jnp.dot(h, Wh_ref[...], preferred_element_type=f32)
        q8 = jnp.concatenate(
            [jnp.dot(S, q, preferred_element_type=f32) for S in SELS],
            axis=1)
        qt = jnp.broadcast_to(q8.reshape(AQ, GB, 1, WL),
                              (AQ, GB, A, WL)).reshape(PQ, WL)
        z2 = jnp.dot(saug, Wr_ref[...], preferred_element_type=f32)
        xm = qt + z2
        wm = xm * (0.7978845608028654 + 0.035677408136300125 * (xm * xm))
        m = xm + xm * jnp.tanh(wm)
        mw = jnp.dot(m, Wua_ref[...], preferred_element_type=f32)
        aggw = jnp.sum(mw.reshape(AQ, M, H), axis=0)
        upd = gelu(jnp.dot(h, Wuh_ref[...], preferred_element_type=f32)
                   + aggw + bu_ref[...])
        h = h + upd

    xg = jnp.sum(h.reshape(GB, A, H), axis=1) * (1.0 / A)
    z = gelu(jnp.dot(xg, W1g_ref[...], preferred_element_type=f32)
             + jnp.dot(mol, W1m_ref[...], preferred_element_type=f32)
             + bf1_ref[...])
    z = gelu(jnp.dot(z, W2_ref[...], preferred_element_type=f32) + bf2_ref[...])
    out_ref[...] = jnp.dot(z, Wo_ref[...], preferred_element_type=f32)


def kernel(x, pos, batch, ptr, aux_ind, num_graphs, atom_emb, W_node, b_node,
           Wh0, Wr0, Wu0, bu0, Wh1, Wr1, Wu1, bu1, Wh2, Wr2, Wu2, bu2,
           W_mol, b_mol, W_fc1, b_fc1, W_fc2, b_fc2, W_out):
    f32 = jnp.float32
    T = jnp.zeros((128, H), f32).at[:NTYPES].set(
        atom_emb @ W_node[:EMB])
    Wn = W_node[EMB:]
    posr = jnp.repeat(pos, P8, axis=1)
    posc = pos.reshape(G // GB, GB, AQ, P8, 3).transpose(0, 2, 1, 4, 3).reshape(
        G // GB, AQ, GB, 3 * P8)

    def blockdiag8(W):
        Z = jnp.zeros((P8 * R + P8, WL), f32)
        for p in range(P8):
            Z = Z.at[p * R:(p + 1) * R, p * F:(p + 1) * F].set(W)
            Z = Z.at[P8 * R + p, p * F:(p + 1) * F].set(1.0)
        return Z

    row_specs = [
        pl.BlockSpec((M, NAF), lambda g: (g, 0)),
        pl.BlockSpec((M, 3 * P8), lambda g: (g, 0)),
        pl.BlockSpec((1, AQ, GB, 3 * P8), lambda g: (g, 0, 0, 0)),
    ]

    full = lambda a: pl.BlockSpec(a.shape, lambda g: tuple(0 for _ in a.shape))
    tile8 = lambda Wua: jnp.tile(0.5 * Wua, (P8, 1))
    weights = [T, Wn, b_node.reshape(1, H),
               Wh0, blockdiag8(Wr0), Wu0[:H], tile8(Wu0[H:]), bu0.reshape(1, H),
               Wh1, blockdiag8(Wr1), Wu1[:H], tile8(Wu1[H:]), bu1.reshape(1, H),
               Wh2, blockdiag8(Wr2), Wu2[:H], tile8(Wu2[H:]), bu2.reshape(1, H),
               W_mol, b_mol.reshape(1, NMF),
               W_fc1[:H], W_fc1[H:], b_fc1.reshape(1, H),
               W_fc2, b_fc2.reshape(1, H), W_out]

    out = pl.pallas_call(
        _block_kernel,
        grid=(G // GB,),
        in_specs=row_specs + [full(w) for w in weights],
        out_specs=pl.BlockSpec((GB, OUT), lambda g: (g, 0)),
        out_shape=jax.ShapeDtypeStruct((G, OUT), f32),
        compiler_params=pltpu.CompilerParams(
            dimension_semantics=("parallel",)),
        interpret=_INTERPRET,
    )(x, posr, posc, *weights)
    return out
